# baseline (speedup 1.0000x reference)
import functools

import jax
import jax.numpy as jnp
from jax import lax
from jax.experimental import pallas as pl
from jax.experimental.pallas import tpu as pltpu
from jax.experimental.pallas import tpu_sc as plsc

VOCAB = 1000000
DIM = 64
B = 1024
L = 200
NW = 32
TOK = B * L
TOK_PER_W = TOK // NW
CHUNK = 128
NCHUNK = TOK_PER_W // CHUNK
NV = DIM // 16

_EPS = 1e-5
_SCALE = 8.0


def _pos_encoding():
    dim_loc = jnp.arange(0, DIM, 2).astype(jnp.float32)
    pos_loc = jnp.arange(0, L).astype(jnp.float32)
    denominator = jnp.exp(-(dim_loc / DIM) * jnp.log(10000.0))
    ang = pos_loc[:, None] * denominator[None, :]
    pe = jnp.zeros((L, DIM), dtype=jnp.float32)
    pe = pe.at[:, 0::2].set(jnp.sin(ang))
    pe = pe.at[:, 1::2].set(jnp.cos(ang))
    return pe


def _rsqrt_newton(v):
    bits = lax.bitcast_convert_type(v, jnp.int32)
    y = lax.bitcast_convert_type(
        jnp.int32(0x5F3759DF) - lax.shift_right_logical(bits, 1), jnp.float32)
    half = 0.5 * v
    for _ in range(3):
        y = y * (1.5 - half * y * y)
    return y


def _worker_id():
    return lax.axis_index("s") * 2 + lax.axis_index("c")


def _sc_body(x_hbm, table_hbm, pe_hbm, g_hbm, b_hbm, out_hbm,
             idx_v, rows_v, pe_v, g_v, b_v, sem):
    wid = _worker_id()
    pltpu.sync_copy(pe_hbm, pe_v)
    pltpu.sync_copy(g_hbm, g_v)
    pltpu.sync_copy(b_hbm, b_v)

    gvec = [g_v[pl.ds(j * 16, 16)] for j in range(NV)]
    bvec = [b_v[pl.ds(j * 16, 16)] for j in range(NV)]

    w_base = wid * TOK_PER_W

    def chunk_body(c, _):
        base = w_base + c * CHUNK
        pltpu.sync_copy(x_hbm.at[pl.ds(base, CHUNK)], idx_v)
        pltpu.async_copy(table_hbm.at[idx_v], rows_v, sem).wait()

        pos0 = lax.rem(c * CHUNK, L)

        def row_body(r, _):
            pos = lax.rem(pos0 + r, L)
            emb = [rows_v[r, pl.ds(j * 16, 16)] * _SCALE
                   + pe_v[pos, pl.ds(j * 16, 16)] for j in range(NV)]
            s = jnp.sum(emb[0] + emb[1] + emb[2] + emb[3])
            ss = jnp.sum(emb[0] * emb[0] + emb[1] * emb[1]
                         + emb[2] * emb[2] + emb[3] * emb[3])
            mean = s * (1.0 / DIM)
            var = ss * (1.0 / DIM) - mean * mean
            rstd = _rsqrt_newton(var + _EPS)
            for j in range(NV):
                rows_v[r, pl.ds(j * 16, 16)] = (
                    (emb[j] - mean) * rstd * gvec[j] + bvec[j])
            return 0

        lax.fori_loop(0, CHUNK, row_body, 0, unroll=8)
        pltpu.sync_copy(rows_v, out_hbm.at[pl.ds(base, CHUNK)])
        return 0

    lax.fori_loop(0, NCHUNK, chunk_body, 0)


def kernel(x, table, ln_gamma, ln_beta):
    x_flat = x.reshape(-1).astype(jnp.int32)
    pe = _pos_encoding()

    mesh = plsc.VectorSubcoreMesh(core_axis_name="c", subcore_axis_name="s")
    run = pl.kernel(
        _sc_body,
        out_type=jax.ShapeDtypeStruct((TOK, DIM), jnp.float32),
        mesh=mesh,
        compiler_params=pltpu.CompilerParams(
            needs_layout_passes=False, use_tc_tiling_on_sc=False),
        scratch_types=[
            pltpu.VMEM((CHUNK,), jnp.int32),
            pltpu.VMEM((CHUNK, DIM), jnp.float32),
            pltpu.VMEM((L, DIM), jnp.float32),
            pltpu.VMEM((DIM,), jnp.float32),
            pltpu.VMEM((DIM,), jnp.float32),
            pltpu.SemaphoreType.DMA,
        ],
    )
    out = run(x_flat, table, pe, ln_gamma, ln_beta)
    return out.reshape(B, L, DIM)


# speedup vs baseline: 1.0707x; 1.0707x over previous
import jax
import jax.numpy as jnp
from jax import lax
from jax.experimental import pallas as pl
from jax.experimental.pallas import tpu as pltpu
from jax.experimental.pallas import tpu_sc as plsc

VOCAB = 1000000
DIM = 64
B = 1024
L = 200
NW = 32
TOK = B * L
TOK_PER_W = TOK // NW
CHUNK = 128
NCHUNK = TOK_PER_W // CHUNK
NBUF = 5
NOUTER = NCHUNK // NBUF
NV = DIM // 16

_EPS = 1e-5
_SCALE = 8.0


def _pos_encoding():
    dim_loc = jnp.arange(0, DIM, 2).astype(jnp.float32)
    pos_loc = jnp.arange(0, L).astype(jnp.float32)
    denominator = jnp.exp(-(dim_loc / DIM) * jnp.log(10000.0))
    ang = pos_loc[:, None] * denominator[None, :]
    pe = jnp.zeros((L, DIM), dtype=jnp.float32)
    pe = pe.at[:, 0::2].set(jnp.sin(ang))
    pe = pe.at[:, 1::2].set(jnp.cos(ang))
    return pe


def _rsqrt_newton(v):
    bits = lax.bitcast_convert_type(v, jnp.int32)
    y = lax.bitcast_convert_type(
        jnp.int32(0x5F3759DF) - lax.shift_right_logical(bits, 1), jnp.float32)
    half = 0.5 * v
    for _ in range(3):
        y = y * (1.5 - half * y * y)
    return y


def _worker_id():
    return lax.axis_index("s") * 2 + lax.axis_index("c")


def _sc_body(x_hbm, table_hbm, pe_hbm, g_hbm, b_hbm, out_hbm,
             idx2d, rows, obuf, pe_v, g_v, b_v, gsem, osem):
    wid = _worker_id()
    pltpu.sync_copy(x_hbm.at[wid], idx2d)
    pltpu.sync_copy(pe_hbm, pe_v)
    pltpu.sync_copy(g_hbm, g_v)
    pltpu.sync_copy(b_hbm, b_v)

    gvec = [g_v[pl.ds(j * 16, 16)] for j in range(NV)]
    bvec = [b_v[pl.ds(j * 16, 16)] for j in range(NV)]

    w_base = wid * TOK_PER_W

    for b in range(NBUF):
        pltpu.make_async_copy(
            table_hbm.at[idx2d.at[b]], rows[b], gsem[b]).start()

    def compute_chunk(c, rbuf, wbuf):
        pos0 = lax.rem(c * CHUNK, L)

        def row_body(r, _):
            pos = lax.rem(pos0 + r, L)
            emb = [rbuf[r, pl.ds(j * 16, 16)] * _SCALE
                   + pe_v[pos, pl.ds(j * 16, 16)] for j in range(NV)]
            s = jnp.sum(emb[0] + emb[1] + emb[2] + emb[3])
            ss = jnp.sum(emb[0] * emb[0] + emb[1] * emb[1]
                         + emb[2] * emb[2] + emb[3] * emb[3])
            mean = s * (1.0 / DIM)
            var = ss * (1.0 / DIM) - mean * mean
            rstd = _rsqrt_newton(var + _EPS)
            for j in range(NV):
                wbuf[r, pl.ds(j * 16, 16)] = (
                    (emb[j] - mean) * rstd * gvec[j] + bvec[j])
            return 0

        lax.fori_loop(0, CHUNK, row_body, 0, unroll=8)

    def outer_body(c0, _):
        for b in range(NBUF):
            c = c0 * NBUF + b
            base = w_base + c * CHUNK
            pltpu.make_async_copy(
                table_hbm.at[idx2d.at[b]], rows[b], gsem[b]).wait()

            @pl.when(c0 > 0)
            def _wait_out():
                pltpu.make_async_copy(
                    obuf[b], out_hbm.at[pl.ds(base, CHUNK)], osem[b]).wait()

            compute_chunk(c, rows[b], obuf[b])

            pltpu.make_async_copy(
                obuf[b], out_hbm.at[pl.ds(base, CHUNK)], osem[b]).start()

            @pl.when(c0 < NOUTER - 1)
            def _fire_next():
                pltpu.make_async_copy(
                    table_hbm.at[idx2d.at[c + NBUF]], rows[b], gsem[b]).start()
        return 0

    lax.fori_loop(0, NOUTER, outer_body, 0)

    for b in range(NBUF):
        c = (NOUTER - 1) * NBUF + b
        base = w_base + c * CHUNK
        pltpu.make_async_copy(
            obuf[b], out_hbm.at[pl.ds(base, CHUNK)], osem[b]).wait()


def kernel(x, table, ln_gamma, ln_beta):
    x3 = x.reshape(NW, NCHUNK, CHUNK).astype(jnp.int32)
    pe = _pos_encoding()

    mesh = plsc.VectorSubcoreMesh(core_axis_name="c", subcore_axis_name="s")
    run = pl.kernel(
        _sc_body,
        out_type=jax.ShapeDtypeStruct((TOK, DIM), jnp.float32),
        mesh=mesh,
        compiler_params=pltpu.CompilerParams(
            needs_layout_passes=False, use_tc_tiling_on_sc=False),
        scratch_types=[
            pltpu.VMEM((NCHUNK, CHUNK), jnp.int32),
            [pltpu.VMEM((CHUNK, DIM), jnp.float32) for _ in range(NBUF)],
            [pltpu.VMEM((CHUNK, DIM), jnp.float32) for _ in range(NBUF)],
            pltpu.VMEM((L, DIM), jnp.float32),
            pltpu.VMEM((DIM,), jnp.float32),
            pltpu.VMEM((DIM,), jnp.float32),
            [pltpu.SemaphoreType.DMA for _ in range(NBUF)],
            [pltpu.SemaphoreType.DMA for _ in range(NBUF)],
        ],
    )
    out = run(x3, table, pe, ln_gamma, ln_beta)
    return out.reshape(B, L, DIM)
